# baseline (speedup 1.0000x reference)
import functools

import jax
import jax.numpy as jnp
from jax import lax
from jax.experimental import pallas as pl
from jax.experimental.pallas import tpu as pltpu
from jax.experimental.pallas import tpu_sc as plsc

N = 10000
E = 320000
F = 128

NC = 2
NS = 16
LANES = 16

NP = 10240
ROWS_PER_TILE = NP // NS
CH = 80
EDGES_PER_TILE = E // NS
N_PHASES = 5
PH_CHUNKS = EDGES_PER_TILE // (N_PHASES * CH)
EDGES_PER_TILE32 = E // (NC * NS)

_mesh = plsc.VectorSubcoreMesh(core_axis_name="c", subcore_axis_name="s")


@functools.partial(
    pl.kernel,
    out_type=jax.ShapeDtypeStruct((NC * NS * N,), jnp.float32),
    mesh=_mesh,
    scratch_types=[
        pltpu.VMEM((N,), jnp.float32),
        pltpu.VMEM((EDGES_PER_TILE32,), jnp.int32),
    ],
    compiler_params=pltpu.CompilerParams(needs_layout_passes=False),
)
def _deg_kernel(col_hbm, out_hbm, deg_buf, idx_buf):
    core = lax.axis_index("c")
    sub = lax.axis_index("s")
    wid = sub * NC + core

    def zero_body(i, _):
        deg_buf[pl.ds(i * LANES, LANES)] = jnp.zeros((LANES,), jnp.float32)
        return 0

    lax.fori_loop(0, N // LANES, zero_body, 0)

    pltpu.sync_copy(col_hbm.at[pl.ds(wid * EDGES_PER_TILE32, EDGES_PER_TILE32)],
                    idx_buf)

    ones = jnp.ones((LANES,), jnp.float32)

    def count_body(i, _):
        v = idx_buf[pl.ds(i * LANES, LANES)]
        plsc.addupdate_scatter(deg_buf, [v], ones)
        return 0

    lax.fori_loop(0, EDGES_PER_TILE32 // LANES, count_body, 0)

    pltpu.sync_copy(deg_buf, out_hbm.at[pl.ds(wid * N, N)])


@functools.partial(
    pl.kernel,
    out_type=(
        jax.ShapeDtypeStruct((NP, F), jnp.float32),
        jax.ShapeDtypeStruct((NP, F), jnp.float32),
    ),
    mesh=_mesh,
    scratch_types=[
        pltpu.VMEM_SHARED((NP, F), jnp.float32),
        pltpu.VMEM((PH_CHUNKS, CH), jnp.int32),
        pltpu.VMEM((PH_CHUNKS, CH), jnp.int32),
        pltpu.VMEM((CH, F), jnp.float32),
        pltpu.VMEM((CH, F), jnp.float32),
        pltpu.SemaphoreType.DMA,
        pltpu.SemaphoreType.DMA,
        pltpu.SemaphoreType.DMA,
        pltpu.SemaphoreType.DMA,
    ],
)
def _spmm_kernel(gp_hbm, gv_hbm, row_hbm, col_hbm, sp_hbm, sv_hbm,
                 accum, ridx, cidx, gbuf0, gbuf1,
                 gsem0, gsem1, ssem0, ssem1):
    core = lax.axis_index("c")
    sub = lax.axis_index("s")

    def zrow(i, _):
        for j in range(F // LANES):
            gbuf0[i, pl.ds(j * LANES, LANES)] = jnp.zeros((LANES,), jnp.float32)
        return 0

    lax.fori_loop(0, CH, zrow, 0)
    for z in range(ROWS_PER_TILE // CH):
        pltpu.sync_copy(
            gbuf0, accum.at[pl.ds(sub * ROWS_PER_TILE + z * CH, CH)])

    plsc.subcore_barrier()

    gbufs = (gbuf0, gbuf1)
    gsems = (gsem0, gsem1)
    ssems = (ssem0, ssem1)

    def edge_loop(g_hbm):
        for ph in range(N_PHASES):
            pltpu.sync_copy(row_hbm.at[sub, ph], ridx)
            pltpu.sync_copy(col_hbm.at[sub, ph], cidx)
            pltpu.async_copy(g_hbm.at[ridx.at[0]], gbuf0, gsem0)

            def chunk(i, b, first, last):
                @pl.when(jnp.logical_and(i >= 1, jnp.logical_not(first)))
                def _():
                    pltpu.make_async_copy(
                        gbufs[1 - b], accum.at[cidx.at[i - 1]],
                        ssems[1 - b]).wait()

                @pl.when(jnp.logical_not(last))
                def _():
                    pltpu.async_copy(g_hbm.at[ridx.at[i + 1]],
                                     gbufs[1 - b], gsems[1 - b])

                pltpu.make_async_copy(g_hbm.at[ridx.at[i]], gbufs[b],
                                      gsems[b]).wait()
                pltpu.async_copy(gbufs[b], accum.at[cidx.at[i]],
                                 ssems[b], add=True)

            def body(g, _):
                for b in range(2):
                    i = 2 * g + b
                    chunk(i, b, i < 1, i + 1 >= PH_CHUNKS)
                return 0

            lax.fori_loop(0, PH_CHUNKS // 2, body, 0)
            pltpu.make_async_copy(gbuf1, accum.at[cidx.at[PH_CHUNKS - 1]],
                                  ssem1).wait()

    @pl.when(core == 0)
    def _():
        edge_loop(gp_hbm)

    @pl.when(core == 1)
    def _():
        edge_loop(gv_hbm)

    plsc.subcore_barrier()

    r0 = sub * ROWS_PER_TILE

    @pl.when(core == 0)
    def _():
        pltpu.sync_copy(accum.at[pl.ds(r0, ROWS_PER_TILE)],
                        sp_hbm.at[pl.ds(r0, ROWS_PER_TILE)])

    @pl.when(core == 1)
    def _():
        pltpu.sync_copy(accum.at[pl.ds(r0, ROWS_PER_TILE)],
                        sv_hbm.at[pl.ds(r0, ROWS_PER_TILE)])


def _matT(a, w):
    return lax.dot_general(a.astype(jnp.bfloat16), w.astype(jnp.bfloat16),
                           (((1,), (1,)), ((), ())),
                           preferred_element_type=jnp.float32)


def _tc1a_body(x_ref, w1_ref, wv1_ref, hp_ref, hv_ref):
    x = x_ref[...]
    hp_ref[...] = _matT(x, w1_ref[...])
    hv_ref[...] = _matT(x, wv1_ref[...])


def _tc1b_body(deg_ref, hp_ref, hv_ref, dinv_ref, gp_ref, gv_ref):
    deg = jnp.sum(deg_ref[...], axis=0, keepdims=True) + 1.0
    dinv_col = lax.rsqrt(deg).T
    dinv_ref[...] = dinv_col
    gp_ref[...] = dinv_col * hp_ref[...]
    gv_ref[...] = dinv_col * hv_ref[...]


def _tc2_body(dinv_ref, sp_ref, gp_ref, b1_ref, w2_ref,
              sv_ref, gv_ref, bv1_ref, wv2_ref, g2p_ref, g2v_ref):
    dinv = dinv_ref[...]
    h1 = jnp.maximum(dinv * (sp_ref[...] + gp_ref[...]) + b1_ref[...], 0.0)
    v1 = jnp.maximum(dinv * (sv_ref[...] + gv_ref[...]) + bv1_ref[...], 0.0)
    g2p_ref[...] = dinv * _matT(h1, w2_ref[...])
    g2v_ref[...] = dinv * _matT(v1, wv2_ref[...])


def _tc3_body(dinv_ref, sp_ref, gp_ref, b2_ref, wp_ref, bp_ref,
              sv_ref, gv_ref, bv2_ref, wvh_ref, bvh_ref, am_ref, val_ref):
    dinv = dinv_ref[...]
    h2 = jnp.maximum(dinv * (sp_ref[...] + gp_ref[...]) + b2_ref[...], 0.0)
    v2 = jnp.maximum(dinv * (sv_ref[...] + gv_ref[...]) + bv2_ref[...], 0.0)
    def _bf(a):
        return a.astype(jnp.bfloat16).astype(jnp.float32)

    am_ref[...] = (jnp.sum(_bf(h2) * _bf(wp_ref[...]), axis=1, keepdims=True)
                   + bp_ref[0, 0])
    val_ref[...] = (jnp.sum(_bf(v2) * _bf(wvh_ref[...]), axis=1, keepdims=True)
                    + bvh_ref[0, 0])


_RB = 2000
_GRID = N // _RB


def _row_spec(i_map=lambda i: (i, 0), shape=(_RB, F)):
    return pl.BlockSpec(shape, i_map)


_tc1a = pl.pallas_call(
    _tc1a_body,
    grid=(_GRID,),
    in_specs=[
        _row_spec(),
        pl.BlockSpec((F, F), lambda i: (0, 0)),
        pl.BlockSpec((F, F), lambda i: (0, 0)),
    ],
    out_specs=[_row_spec(), _row_spec()],
    out_shape=(
        jax.ShapeDtypeStruct((N, F), jnp.float32),
        jax.ShapeDtypeStruct((N, F), jnp.float32),
    ),
)

_tc1b = pl.pallas_call(
    _tc1b_body,
    out_shape=(
        jax.ShapeDtypeStruct((N, 1), jnp.float32),
        jax.ShapeDtypeStruct((N, F), jnp.float32),
        jax.ShapeDtypeStruct((N, F), jnp.float32),
    ),
)

_tc2 = pl.pallas_call(
    _tc2_body,
    grid=(_GRID,),
    in_specs=[
        pl.BlockSpec((_RB, 1), lambda i: (i, 0)),
        _row_spec(), _row_spec(),
        pl.BlockSpec((1, F), lambda i: (0, 0)),
        pl.BlockSpec((F, F), lambda i: (0, 0)),
        _row_spec(), _row_spec(),
        pl.BlockSpec((1, F), lambda i: (0, 0)),
        pl.BlockSpec((F, F), lambda i: (0, 0)),
    ],
    out_specs=[_row_spec(), _row_spec()],
    out_shape=(
        jax.ShapeDtypeStruct((N, F), jnp.float32),
        jax.ShapeDtypeStruct((N, F), jnp.float32),
    ),
)

_tc3 = pl.pallas_call(
    _tc3_body,
    grid=(_GRID,),
    in_specs=[
        pl.BlockSpec((_RB, 1), lambda i: (i, 0)),
        _row_spec(), _row_spec(),
        pl.BlockSpec((1, F), lambda i: (0, 0)),
        pl.BlockSpec((1, F), lambda i: (0, 0)),
        pl.BlockSpec((1, 1), lambda i: (0, 0)),
        _row_spec(), _row_spec(),
        pl.BlockSpec((1, F), lambda i: (0, 0)),
        pl.BlockSpec((1, F), lambda i: (0, 0)),
        pl.BlockSpec((1, 1), lambda i: (0, 0)),
    ],
    out_specs=[
        pl.BlockSpec((_RB, 1), lambda i: (i, 0)),
        pl.BlockSpec((_RB, 1), lambda i: (i, 0)),
    ],
    out_shape=(
        jax.ShapeDtypeStruct((N, 1), jnp.float32),
        jax.ShapeDtypeStruct((N, 1), jnp.float32),
    ),
)


def kernel(x, edge_index, W1, b1, W2, b2, Wp, bp, Wv1, bv1, Wv2, bv2, Wvh, bvh):
    row = edge_index[0].astype(jnp.int32)
    col = edge_index[1].astype(jnp.int32)
    row3 = row.reshape(NS, N_PHASES, PH_CHUNKS, CH)
    col3 = col.reshape(NS, N_PHASES, PH_CHUNKS, CH)
    b1r = jnp.reshape(b1, (1, F))
    b2r = jnp.reshape(b2, (1, F))
    bv1r = jnp.reshape(bv1, (1, F))
    bv2r = jnp.reshape(bv2, (1, F))
    bpr = jnp.reshape(bp, (1, 1))
    bvhr = jnp.reshape(bvh, (1, 1))

    deg_parts = _deg_kernel(col).reshape(NC * NS, N)
    h1p, h1v = _tc1a(x, W1, Wv1)
    dinv, g1p, g1v = _tc1b(deg_parts, h1p, h1v)
    s1p, s1v = _spmm_kernel(g1p, g1v, row3, col3)
    g2p, g2v = _tc2(dinv, s1p, g1p, b1r, W2, s1v, g1v, bv1r, Wv2)
    s2p, s2v = _spmm_kernel(g2p, g2v, row3, col3)
    am, val = _tc3(dinv, s2p, g2p, b2r, Wp, bpr, s2v, g2v, bv2r, Wvh, bvhr)
    return (am, val)


# speedup vs baseline: 1.0163x; 1.0163x over previous
import functools

import jax
import jax.numpy as jnp
from jax import lax
from jax.experimental import pallas as pl
from jax.experimental.pallas import tpu as pltpu
from jax.experimental.pallas import tpu_sc as plsc

N = 10000
E = 320000
F = 128

NC = 2
NS = 16
LANES = 16

NP = 10240
ROWS_PER_TILE = NP // NS
CH = 80
EDGES_PER_TILE = E // NS
N_PHASES = 5
PH_CHUNKS = EDGES_PER_TILE // (N_PHASES * CH)
EDGES_PER_TILE32 = E // (NC * NS)

_mesh = plsc.VectorSubcoreMesh(core_axis_name="c", subcore_axis_name="s")


@functools.partial(
    pl.kernel,
    out_type=jax.ShapeDtypeStruct((NC * NS * N,), jnp.float32),
    mesh=_mesh,
    scratch_types=[
        pltpu.VMEM((N,), jnp.float32),
        pltpu.VMEM((EDGES_PER_TILE32,), jnp.int32),
    ],
    compiler_params=pltpu.CompilerParams(needs_layout_passes=False),
)
def _deg_kernel(col_hbm, out_hbm, deg_buf, idx_buf):
    core = lax.axis_index("c")
    sub = lax.axis_index("s")
    wid = sub * NC + core

    def zero_body(i, _):
        deg_buf[pl.ds(i * LANES, LANES)] = jnp.zeros((LANES,), jnp.float32)
        return 0

    lax.fori_loop(0, N // LANES, zero_body, 0)

    pltpu.sync_copy(col_hbm.at[pl.ds(wid * EDGES_PER_TILE32, EDGES_PER_TILE32)],
                    idx_buf)

    ones = jnp.ones((LANES,), jnp.float32)

    def count_body(i, _):
        v = idx_buf[pl.ds(i * LANES, LANES)]
        plsc.addupdate_scatter(deg_buf, [v], ones)
        return 0

    lax.fori_loop(0, EDGES_PER_TILE32 // LANES, count_body, 0)

    pltpu.sync_copy(deg_buf, out_hbm.at[pl.ds(wid * N, N)])


@functools.partial(
    pl.kernel,
    out_type=(
        jax.ShapeDtypeStruct((NP, F), jnp.float32),
        jax.ShapeDtypeStruct((NP, F), jnp.float32),
    ),
    mesh=_mesh,
    scratch_types=[
        pltpu.VMEM_SHARED((NP, F), jnp.float32),
        pltpu.VMEM((PH_CHUNKS, CH), jnp.int32),
        pltpu.VMEM((PH_CHUNKS, CH), jnp.int32),
        pltpu.VMEM((CH, F), jnp.float32),
        pltpu.VMEM((CH, F), jnp.float32),
        pltpu.SemaphoreType.DMA,
        pltpu.SemaphoreType.DMA,
        pltpu.SemaphoreType.DMA,
        pltpu.SemaphoreType.DMA,
    ],
)
def _spmm_kernel(gp_hbm, gv_hbm, row_hbm, col_hbm, sp_hbm, sv_hbm,
                 accum, ridx, cidx, gbuf0, gbuf1,
                 gsem0, gsem1, ssem0, ssem1):
    core = lax.axis_index("c")
    sub = lax.axis_index("s")

    def zrow(i, _):
        for j in range(F // LANES):
            gbuf0[i, pl.ds(j * LANES, LANES)] = jnp.zeros((LANES,), jnp.float32)
        return 0

    lax.fori_loop(0, CH, zrow, 0)
    for z in range(ROWS_PER_TILE // CH):
        pltpu.sync_copy(
            gbuf0, accum.at[pl.ds(sub * ROWS_PER_TILE + z * CH, CH)])

    plsc.subcore_barrier()

    gbufs = (gbuf0, gbuf1)
    gsems = (gsem0, gsem1)
    ssems = (ssem0, ssem1)

    def edge_loop(g_hbm):
        for ph in range(N_PHASES):
            pltpu.sync_copy(row_hbm.at[sub, ph], ridx)
            pltpu.sync_copy(col_hbm.at[sub, ph], cidx)
            pltpu.async_copy(g_hbm.at[ridx.at[0]], gbuf0, gsem0)

            def chunk(i, b, first, last):
                @pl.when(jnp.logical_and(i >= 1, jnp.logical_not(first)))
                def _():
                    pltpu.make_async_copy(
                        gbufs[1 - b], accum.at[cidx.at[i - 1]],
                        ssems[1 - b]).wait()

                @pl.when(jnp.logical_not(last))
                def _():
                    pltpu.async_copy(g_hbm.at[ridx.at[i + 1]],
                                     gbufs[1 - b], gsems[1 - b])

                pltpu.make_async_copy(g_hbm.at[ridx.at[i]], gbufs[b],
                                      gsems[b]).wait()
                pltpu.async_copy(gbufs[b], accum.at[cidx.at[i]],
                                 ssems[b], add=True)

            def body(g, _):
                for b in range(2):
                    i = 2 * g + b
                    chunk(i, b, i < 1, i + 1 >= PH_CHUNKS)
                return 0

            lax.fori_loop(0, PH_CHUNKS // 2, body, 0)
            pltpu.make_async_copy(gbuf1, accum.at[cidx.at[PH_CHUNKS - 1]],
                                  ssem1).wait()

    @pl.when(core == 0)
    def _():
        edge_loop(gp_hbm)

    @pl.when(core == 1)
    def _():
        edge_loop(gv_hbm)

    plsc.subcore_barrier()

    r0 = sub * ROWS_PER_TILE

    @pl.when(core == 0)
    def _():
        pltpu.sync_copy(accum.at[pl.ds(r0, ROWS_PER_TILE)],
                        sp_hbm.at[pl.ds(r0, ROWS_PER_TILE)])

    @pl.when(core == 1)
    def _():
        pltpu.sync_copy(accum.at[pl.ds(r0, ROWS_PER_TILE)],
                        sv_hbm.at[pl.ds(r0, ROWS_PER_TILE)])


def _matT(a, w):
    return lax.dot_general(a.astype(jnp.bfloat16), w.astype(jnp.bfloat16),
                           (((1,), (1,)), ((), ())),
                           preferred_element_type=jnp.float32)


def _tc1_body(deg_ref, x_ref, w1_ref, wv1_ref, dinv_ref, gp_ref, gv_ref):
    deg = jnp.sum(deg_ref[...], axis=0, keepdims=True) + 1.0
    dinv_col = lax.rsqrt(deg).T
    dinv_ref[...] = dinv_col
    x = x_ref[...]
    gp_ref[...] = dinv_col * _matT(x, w1_ref[...])
    gv_ref[...] = dinv_col * _matT(x, wv1_ref[...])


def _tc2_body(dinv_ref, sp_ref, gp_ref, b1_ref, w2_ref,
              sv_ref, gv_ref, bv1_ref, wv2_ref, g2p_ref, g2v_ref):
    dinv = dinv_ref[...]
    h1 = jnp.maximum(dinv * (sp_ref[...] + gp_ref[...]) + b1_ref[...], 0.0)
    v1 = jnp.maximum(dinv * (sv_ref[...] + gv_ref[...]) + bv1_ref[...], 0.0)
    g2p_ref[...] = dinv * _matT(h1, w2_ref[...])
    g2v_ref[...] = dinv * _matT(v1, wv2_ref[...])


def _tc3_body(dinv_ref, sp_ref, gp_ref, b2_ref, wp_ref, bp_ref,
              sv_ref, gv_ref, bv2_ref, wvh_ref, bvh_ref, am_ref, val_ref):
    dinv = dinv_ref[...]
    h2 = jnp.maximum(dinv * (sp_ref[...] + gp_ref[...]) + b2_ref[...], 0.0)
    v2 = jnp.maximum(dinv * (sv_ref[...] + gv_ref[...]) + bv2_ref[...], 0.0)
    def _bf(a):
        return a.astype(jnp.bfloat16).astype(jnp.float32)

    am_ref[...] = (jnp.sum(_bf(h2) * _bf(wp_ref[...]), axis=1, keepdims=True)
                   + bp_ref[0, 0])
    val_ref[...] = (jnp.sum(_bf(v2) * _bf(wvh_ref[...]), axis=1, keepdims=True)
                    + bvh_ref[0, 0])


_RB = 2000
_GRID = N // _RB


def _row_spec(i_map=lambda i: (i, 0), shape=(_RB, F)):
    return pl.BlockSpec(shape, i_map)


_tc1 = pl.pallas_call(
    _tc1_body,
    out_shape=(
        jax.ShapeDtypeStruct((N, 1), jnp.float32),
        jax.ShapeDtypeStruct((N, F), jnp.float32),
        jax.ShapeDtypeStruct((N, F), jnp.float32),
    ),
)

_tc2 = pl.pallas_call(
    _tc2_body,
    grid=(_GRID,),
    in_specs=[
        pl.BlockSpec((_RB, 1), lambda i: (i, 0)),
        _row_spec(), _row_spec(),
        pl.BlockSpec((1, F), lambda i: (0, 0)),
        pl.BlockSpec((F, F), lambda i: (0, 0)),
        _row_spec(), _row_spec(),
        pl.BlockSpec((1, F), lambda i: (0, 0)),
        pl.BlockSpec((F, F), lambda i: (0, 0)),
    ],
    out_specs=[_row_spec(), _row_spec()],
    out_shape=(
        jax.ShapeDtypeStruct((N, F), jnp.float32),
        jax.ShapeDtypeStruct((N, F), jnp.float32),
    ),
)

_tc3 = pl.pallas_call(
    _tc3_body,
    grid=(_GRID,),
    in_specs=[
        pl.BlockSpec((_RB, 1), lambda i: (i, 0)),
        _row_spec(), _row_spec(),
        pl.BlockSpec((1, F), lambda i: (0, 0)),
        pl.BlockSpec((1, F), lambda i: (0, 0)),
        pl.BlockSpec((1, 1), lambda i: (0, 0)),
        _row_spec(), _row_spec(),
        pl.BlockSpec((1, F), lambda i: (0, 0)),
        pl.BlockSpec((1, F), lambda i: (0, 0)),
        pl.BlockSpec((1, 1), lambda i: (0, 0)),
    ],
    out_specs=[
        pl.BlockSpec((_RB, 1), lambda i: (i, 0)),
        pl.BlockSpec((_RB, 1), lambda i: (i, 0)),
    ],
    out_shape=(
        jax.ShapeDtypeStruct((N, 1), jnp.float32),
        jax.ShapeDtypeStruct((N, 1), jnp.float32),
    ),
)


def kernel(x, edge_index, W1, b1, W2, b2, Wp, bp, Wv1, bv1, Wv2, bv2, Wvh, bvh):
    row = edge_index[0].astype(jnp.int32)
    col = edge_index[1].astype(jnp.int32)
    row3 = row.reshape(NS, N_PHASES, PH_CHUNKS, CH)
    col3 = col.reshape(NS, N_PHASES, PH_CHUNKS, CH)
    b1r = jnp.reshape(b1, (1, F))
    b2r = jnp.reshape(b2, (1, F))
    bv1r = jnp.reshape(bv1, (1, F))
    bv2r = jnp.reshape(bv2, (1, F))
    bpr = jnp.reshape(bp, (1, 1))
    bvhr = jnp.reshape(bvh, (1, 1))

    deg_parts = _deg_kernel(col).reshape(NC * NS, N)
    dinv, g1p, g1v = _tc1(deg_parts, x, W1, Wv1)
    s1p, s1v = _spmm_kernel(g1p, g1v, row3, col3)
    g2p, g2v = _tc2(dinv, s1p, g1p, b1r, W2, s1v, g1v, bv1r, Wv2)
    s2p, s2v = _spmm_kernel(g2p, g2v, row3, col3)
    am, val = _tc3(dinv, s2p, g2p, b2r, Wp, bpr, s2v, g2v, bv2r, Wvh, bvhr)
    return (am, val)
